# baseline (speedup 1.0000x reference)
import functools

import jax
import jax.numpy as jnp
from jax import lax
from jax.experimental import pallas as pl
from jax.experimental.pallas import tpu as pltpu
from jax.experimental.pallas import tpu_sc as plsc

N_VOCAB = 1000000
N_EMBED = 64
N_ROWS = 4096
N_POS = 200

_NC = 2
_NS = 16
_NW = _NC * _NS
_BLK = N_ROWS // _NW

_mesh = plsc.VectorSubcoreMesh(core_axis_name="c", subcore_axis_name="s")

_out5 = jax.ShapeDtypeStruct((N_POS, 8, _NW, 8, _BLK), jnp.float32)


@functools.partial(
    pl.kernel,
    mesh=_mesh,
    compiler_params=pltpu.CompilerParams(
        use_tc_tiling_on_sc=False, needs_layout_passes=False,
        disable_bounds_checks=True),
    out_type=(_out5, _out5),
    scratch_types=[
        pltpu.VMEM((_BLK, N_POS), jnp.int32),
        pltpu.VMEM((N_POS, _BLK), jnp.int32),
        pltpu.VMEM((N_POS, _BLK), jnp.int32),
        pltpu.VMEM((_BLK, 2 * N_EMBED), jnp.float32),
        pltpu.VMEM((_BLK, 2 * N_EMBED), jnp.float32),
        pltpu.VMEM((8, 8, _BLK), jnp.float32),
        pltpu.VMEM((8, 8, _BLK), jnp.float32),
        pltpu.SemaphoreType.DMA,
        pltpu.SemaphoreType.DMA,
        pltpu.SemaphoreType.DMA,
        pltpu.SemaphoreType.DMA,
        pltpu.SemaphoreType.DMA,
        pltpu.SemaphoreType.DMA,
    ],
)
def _embed_gather(x_hbm, table_hbm, outa_hbm, outb_hbm,
                  xt, xp, xh, r0, r1, t0, t1, g0, g1, a0, a1, b0, b1):
    rows = (r0, r1)
    rt = (t0, t1)
    gsem = (g0, g1)
    asem = (a0, a1)
    bsem = (b0, b1)
    wid = lax.axis_index("s") * _NC + lax.axis_index("c")

    pltpu.sync_copy(x_hbm.at[pl.ds(wid * _BLK, _BLK)], xt)
    lane = lax.iota(jnp.int32, 16)

    def xpose_body(p, _):
        for t8 in range(_BLK // 16):
            toks = lane + (t8 * 16)
            vals = plsc.load_gather(xt, [toks, jnp.full((16,), p, jnp.int32)])
            xp[p, pl.ds(t8 * 16, 16)] = vals >> 1
            xh[p, pl.ds(t8 * 16, 16)] = (vals & 1) << 6
        return ()

    lax.fori_loop(0, N_POS, xpose_body, (), unroll=False)

    def g_start(b, p):
        pltpu.async_copy(table_hbm.at[xp.at[p]], rows[b], gsem[b])

    def g_wait(b):
        pltpu.make_async_copy(table_hbm.at[xp.at[0]], rows[b], gsem[b]).wait()

    def w_start(b, p):
        pltpu.async_copy(rt[b], outa_hbm.at[p, :, wid], asem[b])
        pltpu.async_copy(rt[b], outb_hbm.at[p, :, wid], bsem[b])

    def w_wait(b):
        pltpu.make_async_copy(rt[b], outa_hbm.at[0, :, wid], asem[b]).wait()
        pltpu.make_async_copy(rt[b], outb_hbm.at[0, :, wid], bsem[b]).wait()

    toks16 = [lane + (t8 * 16) for t8 in range(_BLK // 16)]

    qe0 = [(lane + q * 16) >> 3 for q in range(4)]
    qe1 = [(lane + q * 16) & 7 for q in range(4)]

    def xpose_block(b, p):
        @plsc.parallel_loop(0, _BLK // 16, unroll=1)
        def g_body(g):
            halfvec = xh[p, pl.ds(g * 16, 16)]
            for i in range(16):
                t = g * 16 + i
                half = halfvec[i]
                tvec = jnp.full((16,), t, jnp.int32)
                for q in range(4):
                    v = rows[b][t, pl.ds(half + q * 16, 16)]
                    plsc.store_scatter(rt[b], [qe0[q], qe1[q], tvec], v)

    for b in range(2):
        g_start(b, b)

    for b in range(2):
        p = b
        g_wait(b)
        xpose_block(b, p)
        g_start(b, p + 2)
        w_start(b, p)

    def steady(i, _):
        for b in range(2):
            p = 2 + i * 2 + b
            g_wait(b)
            xpose_block(b, p)
            g_start(b, p + 2)
            w_wait(b)
            w_start(b, p)
        return ()

    lax.fori_loop(0, (N_POS - 4) // 2, steady, (), unroll=False)

    for b in range(2):
        p = N_POS - 2 + b
        g_wait(b)
        xpose_block(b, p)
        w_wait(b)
        w_start(b, p)
    for b in range(2):
        w_wait(b)


def kernel(x, table):
    tbl2 = table.reshape(N_VOCAB // 2, 2 * N_EMBED)
    outa, outb = _embed_gather(x, tbl2)

    def to_logical(o5):
        return o5.transpose(2, 4, 0, 1, 3).reshape(N_ROWS, N_POS, N_EMBED)

    return (to_logical(outa), to_logical(outb))


# speedup vs baseline: 1.3908x; 1.3908x over previous
import functools

import jax
import jax.numpy as jnp
from jax import lax
from jax.experimental import pallas as pl
from jax.experimental.pallas import tpu as pltpu
from jax.experimental.pallas import tpu_sc as plsc

N_VOCAB = 1000000
N_EMBED = 64
N_ROWS = 4096
N_POS = 200

_NC = 2
_NS = 16
_NW = _NC * _NS
_BLK = N_ROWS // _NW

_mesh = plsc.VectorSubcoreMesh(core_axis_name="c", subcore_axis_name="s")

_out5 = jax.ShapeDtypeStruct((N_POS, 8, _NW, 8, _BLK), jnp.float32)


@functools.partial(
    pl.kernel,
    mesh=_mesh,
    compiler_params=pltpu.CompilerParams(
        use_tc_tiling_on_sc=False, needs_layout_passes=False,
        disable_bounds_checks=True),
    out_type=(_out5, _out5),
    scratch_types=[
        pltpu.VMEM((_BLK, N_POS), jnp.int32),
        pltpu.VMEM((N_POS, _BLK), jnp.int32),
        pltpu.VMEM((N_POS, _BLK), jnp.int32),
        pltpu.VMEM((_BLK, 2 * N_EMBED), jnp.float32),
        pltpu.VMEM((_BLK, 2 * N_EMBED), jnp.float32),
        pltpu.VMEM((8, 8, _BLK + 1), jnp.float32),
        pltpu.VMEM((8, 8, _BLK + 1), jnp.float32),
        pltpu.SemaphoreType.DMA,
        pltpu.SemaphoreType.DMA,
        pltpu.SemaphoreType.DMA,
        pltpu.SemaphoreType.DMA,
        pltpu.SemaphoreType.DMA,
        pltpu.SemaphoreType.DMA,
    ],
)
def _embed_gather(x_hbm, table_hbm, outa_hbm, outb_hbm,
                  xt, xp, xh, r0, r1, t0, t1, g0, g1, a0, a1, b0, b1):
    rows = (r0, r1)
    rt = (t0, t1)
    gsem = (g0, g1)
    asem = (a0, a1)
    bsem = (b0, b1)
    wid = lax.axis_index("s") * _NC + lax.axis_index("c")

    pltpu.sync_copy(x_hbm.at[pl.ds(wid * _BLK, _BLK)], xt)
    lane = lax.iota(jnp.int32, 16)

    def xpose_body(p, _):
        for t8 in range(_BLK // 16):
            toks = lane + (t8 * 16)
            vals = plsc.load_gather(xt, [toks, jnp.full((16,), p, jnp.int32)])
            xp[p, pl.ds(t8 * 16, 16)] = vals >> 1
            xh[p, pl.ds(t8 * 16, 16)] = (vals & 1) << 6
        return ()

    lax.fori_loop(0, N_POS, xpose_body, (), unroll=False)

    def g_start(b, p):
        pltpu.async_copy(table_hbm.at[xp.at[p]], rows[b], gsem[b])

    def g_wait(b):
        pltpu.make_async_copy(table_hbm.at[xp.at[0]], rows[b], gsem[b]).wait()

    def w_start(b, p):
        pltpu.async_copy(rt[b].at[:, :, pl.ds(0, _BLK)], outa_hbm.at[p, :, wid], asem[b])
        pltpu.async_copy(rt[b].at[:, :, pl.ds(0, _BLK)], outb_hbm.at[p, :, wid], bsem[b])

    def w_wait(b):
        pltpu.make_async_copy(rt[b].at[:, :, pl.ds(0, _BLK)], outa_hbm.at[0, :, wid], asem[b]).wait()
        pltpu.make_async_copy(rt[b].at[:, :, pl.ds(0, _BLK)], outb_hbm.at[0, :, wid], bsem[b]).wait()

    toks16 = [lane + (t8 * 16) for t8 in range(_BLK // 16)]

    qe0 = [(lane + q * 16) >> 3 for q in range(4)]
    qe1 = [(lane + q * 16) & 7 for q in range(4)]

    def xpose_block(b, p):
        @plsc.parallel_loop(0, _BLK // 16, unroll=1)
        def g_body(g):
            halfvec = xh[p, pl.ds(g * 16, 16)]
            for i in range(16):
                t = g * 16 + i
                half = halfvec[i]
                tvec = jnp.full((16,), t, jnp.int32)
                for q in range(4):
                    v = rows[b][t, pl.ds(half + q * 16, 16)]
                    plsc.store_scatter(rt[b], [qe0[q], qe1[q], tvec], v)

    for b in range(2):
        g_start(b, b)

    for b in range(2):
        p = b
        g_wait(b)
        xpose_block(b, p)
        g_start(b, p + 2)
        w_start(b, p)

    def steady(i, _):
        for b in range(2):
            p = 2 + i * 2 + b
            g_wait(b)
            xpose_block(b, p)
            g_start(b, p + 2)
            w_wait(b)
            w_start(b, p)
        return ()

    lax.fori_loop(0, (N_POS - 4) // 2, steady, (), unroll=False)

    for b in range(2):
        p = N_POS - 2 + b
        g_wait(b)
        xpose_block(b, p)
        w_wait(b)
        w_start(b, p)
    for b in range(2):
        w_wait(b)


def kernel(x, table):
    tbl2 = table.reshape(N_VOCAB // 2, 2 * N_EMBED)
    outa, outb = _embed_gather(x, tbl2)

    def to_logical(o5):
        return o5.transpose(2, 4, 0, 1, 3).reshape(N_ROWS, N_POS, N_EMBED)

    return (to_logical(outa), to_logical(outb))


# speedup vs baseline: 1.5312x; 1.1010x over previous
import functools

import jax
import jax.numpy as jnp
from jax import lax
from jax.experimental import pallas as pl
from jax.experimental.pallas import tpu as pltpu
from jax.experimental.pallas import tpu_sc as plsc

N_VOCAB = 1000000
N_EMBED = 64
N_ROWS = 4096
N_POS = 200

_NC = 2
_NS = 16
_NW = _NC * _NS
_BLK = N_ROWS // _NW

_mesh = plsc.VectorSubcoreMesh(core_axis_name="c", subcore_axis_name="s")

_out5 = jax.ShapeDtypeStruct((N_POS, 8, _NW, 8, _BLK), jnp.float32)


@functools.partial(
    pl.kernel,
    mesh=_mesh,
    compiler_params=pltpu.CompilerParams(
        use_tc_tiling_on_sc=False, needs_layout_passes=False,
        disable_bounds_checks=True),
    out_type=(_out5, _out5),
    scratch_types=[
        pltpu.VMEM((_BLK, N_POS), jnp.int32),
        pltpu.VMEM((N_POS, _BLK), jnp.int32),
        pltpu.VMEM((_BLK, 2 * N_EMBED), jnp.float32),
        pltpu.VMEM((_BLK, 2 * N_EMBED), jnp.float32),
        pltpu.VMEM((8, 8, _BLK + 1), jnp.float32),
        pltpu.VMEM((8, 8, _BLK + 1), jnp.float32),
        pltpu.SemaphoreType.DMA,
        pltpu.SemaphoreType.DMA,
        pltpu.SemaphoreType.DMA,
        pltpu.SemaphoreType.DMA,
        pltpu.SemaphoreType.DMA,
        pltpu.SemaphoreType.DMA,
    ],
)
def _embed_gather(x_hbm, table_hbm, outa_hbm, outb_hbm,
                  xt, xp, r0, r1, t0, t1, g0, g1, a0, a1, b0, b1):
    rows = (r0, r1)
    rt = (t0, t1)
    gsem = (g0, g1)
    asem = (a0, a1)
    bsem = (b0, b1)
    wid = lax.axis_index("s") * _NC + lax.axis_index("c")

    pltpu.sync_copy(x_hbm.at[pl.ds(wid * _BLK, _BLK)], xt)
    lane = lax.iota(jnp.int32, 16)

    def xpose_body(p, _):
        for t8 in range(_BLK // 16):
            toks = lane + (t8 * 16)
            vals = plsc.load_gather(xt, [toks, jnp.full((16,), p, jnp.int32)])
            xp[p, pl.ds(t8 * 16, 16)] = vals
        return ()

    lax.fori_loop(0, N_POS, xpose_body, (), unroll=False)

    def g_start(b, p):
        pltpu.async_copy(table_hbm.at[xp.at[p]], rows[b], gsem[b])

    def g_wait(b):
        pltpu.make_async_copy(table_hbm.at[xp.at[0]], rows[b], gsem[b]).wait()

    def w_start(b, p):
        pltpu.async_copy(rt[b].at[:, :, pl.ds(0, _BLK)],
                         outa_hbm.at[p, :, wid], asem[b])
        pltpu.async_copy(rt[b].at[:, :, pl.ds(0, _BLK)],
                         outb_hbm.at[p, :, wid], bsem[b])

    def w_wait(b):
        pltpu.make_async_copy(rt[b].at[:, :, pl.ds(0, _BLK)],
                              outa_hbm.at[0, :, wid], asem[b]).wait()
        pltpu.make_async_copy(rt[b].at[:, :, pl.ds(0, _BLK)],
                              outb_hbm.at[0, :, wid], bsem[b]).wait()

    qe0 = [(lane + q * 16) >> 3 for q in range(4)]
    qe1 = [(lane + q * 16) & 7 for q in range(4)]

    def xpose_block(b):
        @plsc.parallel_loop(0, _BLK // 16, unroll=2)
        def g_body(g):
            for i in range(16):
                t = g * 16 + i
                tvec = jnp.full((16,), t, jnp.int32)
                for q in range(4):
                    v = rows[b][t, pl.ds(q * 16, 16)]
                    plsc.store_scatter(rt[b], [qe0[q], qe1[q], tvec], v)

    for b in range(2):
        g_start(b, b)

    for b in range(2):
        p = b
        g_wait(b)
        xpose_block(b)
        g_start(b, p + 2)
        w_start(b, p)

    def steady(i, _):
        for b in range(2):
            p = 2 + i * 2 + b
            g_wait(b)
            xpose_block(b)
            g_start(b, p + 2)
            w_wait(b)
            w_start(b, p)
        return ()

    lax.fori_loop(0, (N_POS - 4) // 2, steady, (), unroll=False)

    for b in range(2):
        p = N_POS - 2 + b
        g_wait(b)
        xpose_block(b)
        w_wait(b)
        w_start(b, p)
    for b in range(2):
        w_wait(b)


def kernel(x, table):
    tblp = jnp.pad(table, ((0, 0), (0, N_EMBED)))
    outa, outb = _embed_gather(x, tblp)

    def to_logical(o5):
        return o5.transpose(2, 4, 0, 1, 3).reshape(N_ROWS, N_POS, N_EMBED)

    return (to_logical(outa), to_logical(outb))
